# baseline (device time: 60491 ns/iter reference)
import jax
import jax.numpy as jnp
from jax import lax
from jax.experimental import pallas as pl
from jax.experimental.pallas import tpu as pltpu


def kernel(x, W):
    t, d = x.shape
    _, v_half = W.shape

    def body(x_ref, w_ref, out_ref, send_buf, recv_buf, send_sem, recv_sem):
        my_x = lax.axis_index("x")
        my_y = lax.axis_index("y")
        my_z = lax.axis_index("z")
        peer_z = 1 - my_z

        barrier_sem = pltpu.get_barrier_semaphore()
        pl.semaphore_signal(
            barrier_sem, inc=1,
            device_id=(my_x, my_y, peer_z),
            device_id_type=pl.DeviceIdType.MESH,
        )
        pl.semaphore_wait(barrier_sem, 1)

        local = jnp.dot(x_ref[:, :], w_ref[:, :],
                        preferred_element_type=jnp.float32)
        send_buf[:, :] = local

        rdma = pltpu.make_async_remote_copy(
            src_ref=send_buf,
            dst_ref=recv_buf,
            send_sem=send_sem,
            recv_sem=recv_sem,
            device_id=(my_x, my_y, peer_z),
            device_id_type=pl.DeviceIdType.MESH,
        )
        rdma.start()
        rdma.wait()

        peer = recv_buf[:, :]
        m = jnp.maximum(
            jnp.max(local, axis=-1, keepdims=True),
            jnp.max(peer, axis=-1, keepdims=True),
        )
        e_local = jnp.exp(local - m)
        e_peer = jnp.exp(peer - m)
        s = (jnp.sum(e_local, axis=-1, keepdims=True)
             + jnp.sum(e_peer, axis=-1, keepdims=True))
        out_ref[:, pl.ds(my_z * v_half, v_half)] = e_local / s
        out_ref[:, pl.ds(peer_z * v_half, v_half)] = e_peer / s

    return pl.pallas_call(
        body,
        out_shape=jax.ShapeDtypeStruct((t, 2 * v_half), jnp.float32),
        in_specs=[
            pl.BlockSpec(memory_space=pltpu.VMEM),
            pl.BlockSpec(memory_space=pltpu.VMEM),
        ],
        out_specs=pl.BlockSpec(memory_space=pltpu.VMEM),
        scratch_shapes=[
            pltpu.VMEM((t, v_half), jnp.float32),
            pltpu.VMEM((t, v_half), jnp.float32),
            pltpu.SemaphoreType.DMA,
            pltpu.SemaphoreType.DMA,
        ],
        compiler_params=pltpu.CompilerParams(collective_id=0),
    )(x, W)


# device time: 58272 ns/iter; 1.0381x vs baseline; 1.0381x over previous
import jax
import jax.numpy as jnp
from jax import lax
from jax.experimental import pallas as pl
from jax.experimental.pallas import tpu as pltpu

N_CHUNKS = 8


def kernel(x, W):
    t, d = x.shape
    _, v_half = W.shape
    chunk = v_half // N_CHUNKS

    def body(x_ref, w_ref, out_ref, send_buf, recv_buf, send_sems, recv_sems):
        my_x = lax.axis_index("x")
        my_y = lax.axis_index("y")
        my_z = lax.axis_index("z")
        peer_z = 1 - my_z

        barrier_sem = pltpu.get_barrier_semaphore()
        pl.semaphore_signal(
            barrier_sem, inc=1,
            device_id=(my_x, my_y, peer_z),
            device_id_type=pl.DeviceIdType.MESH,
        )
        pl.semaphore_wait(barrier_sem, 1)

        rdmas = []
        for c in range(N_CHUNKS):
            cols = slice(c * chunk, (c + 1) * chunk)
            send_buf[:, cols] = jnp.dot(
                x_ref[:, :], w_ref[:, cols],
                preferred_element_type=jnp.float32,
            )
            rdma = pltpu.make_async_remote_copy(
                src_ref=send_buf.at[:, cols],
                dst_ref=recv_buf.at[:, cols],
                send_sem=send_sems.at[c],
                recv_sem=recv_sems.at[c],
                device_id=(my_x, my_y, peer_z),
                device_id_type=pl.DeviceIdType.MESH,
            )
            rdma.start()
            rdmas.append(rdma)

        local = send_buf[:, :]
        m_l = jnp.max(local, axis=-1, keepdims=True)
        e_l = jnp.exp(local - m_l)
        s_l = jnp.sum(e_l, axis=-1, keepdims=True)
        out_ref[:, pl.ds(my_z * v_half, v_half)] = e_l

        m_cs = []
        s_cs = []
        for c in range(N_CHUNKS):
            rdmas[c].wait_recv()
            cols = slice(c * chunk, (c + 1) * chunk)
            blk = recv_buf[:, cols]
            m_c = jnp.max(blk, axis=-1, keepdims=True)
            e_c = jnp.exp(blk - m_c)
            s_c = jnp.sum(e_c, axis=-1, keepdims=True)
            out_ref[:, pl.ds(peer_z * v_half + c * chunk, chunk)] = e_c
            m_cs.append(m_c)
            s_cs.append(s_c)

        m = m_l
        for m_c in m_cs:
            m = jnp.maximum(m, m_c)
        s = s_l * jnp.exp(m_l - m)
        for m_c, s_c in zip(m_cs, s_cs):
            s = s + s_c * jnp.exp(m_c - m)
        inv = 1.0 / s

        corr_l = jnp.exp(m_l - m) * inv
        loc = pl.ds(my_z * v_half, v_half)
        out_ref[:, loc] = out_ref[:, loc] * corr_l
        for c in range(N_CHUNKS):
            corr_c = jnp.exp(m_cs[c] - m) * inv
            pc = pl.ds(peer_z * v_half + c * chunk, chunk)
            out_ref[:, pc] = out_ref[:, pc] * corr_c

        for rdma in rdmas:
            rdma.wait_send()

    return pl.pallas_call(
        body,
        out_shape=jax.ShapeDtypeStruct((t, 2 * v_half), jnp.float32),
        in_specs=[
            pl.BlockSpec(memory_space=pltpu.VMEM),
            pl.BlockSpec(memory_space=pltpu.VMEM),
        ],
        out_specs=pl.BlockSpec(memory_space=pltpu.VMEM),
        scratch_shapes=[
            pltpu.VMEM((t, v_half), jnp.float32),
            pltpu.VMEM((t, v_half), jnp.float32),
            pltpu.SemaphoreType.DMA((N_CHUNKS,)),
            pltpu.SemaphoreType.DMA((N_CHUNKS,)),
        ],
        compiler_params=pltpu.CompilerParams(collective_id=0),
    )(x, W)
